# Initial kernel scaffold; baseline (speedup 1.0000x reference)
import jax
import jax.numpy as jnp
from jax.experimental import pallas as pl


def kernel(x, centroids_vector, rotors):
    raise NotImplementedError("write your pallas kernel here")


# speedup vs baseline: 1.0297x; 1.0297x over previous
import functools

import numpy as np
import jax
import jax.numpy as jnp
from jax import lax
from jax.experimental import pallas as pl
from jax.experimental.pallas import tpu as pltpu
from jax.experimental.pallas import tpu_sc as plsc

D = 512
NG = (D + 2) // 3
P = NG * 3
PP = 528
Q = NG * 4
QQ = 688
K = 64
NC, NS, L = 2, 16, 16
NW = NC * NS
R = 16


def _gp_table() -> np.ndarray:
    masks = [0b000, 0b001, 0b010, 0b100, 0b011, 0b101, 0b110, 0b111]
    idx = {m: i for i, m in enumerate(masks)}
    T = np.zeros((8, 8, 8), dtype=np.float32)
    for i, a in enumerate(masks):
        for j, b in enumerate(masks):
            t = a >> 1
            swaps = 0
            while t:
                swaps += bin(t & b).count('1')
                t >>= 1
            T[i, j, idx[a ^ b]] = -1.0 if (swaps & 1) else 1.0
    return T


_T = _gp_table()
_REV = np.array([1, 1, 1, 1, -1, -1, -1, -1], dtype=np.float32)
_KMAP = np.array([1, 2, 3, 7])

_J = np.arange(P)
_JG = _J // 3
_JR = _J % 3
_I0 = (3 * _JG).astype(np.int32)
_K4G = (np.arange(Q) // 4)
_K4M = (np.arange(Q) % 4)


def _padp(v):
    return jnp.pad(v, (0, PP - P))


def _padq(v):
    return jnp.pad(v, (0, QQ - Q))


def _rbf_host(v):
    return v.astype(jnp.bfloat16).astype(jnp.float32)


def _coef_tables(rotors):
    rot = rotors.astype(jnp.float32)
    rr = rot * _REV
    T = jnp.asarray(_T)
    hi = lax.Precision.HIGHEST
    w1 = jnp.einsum('gi,ijk->gjk', rot, T, precision=hi)
    w2 = jnp.einsum('gj,ijk->gik', rr, T, precision=hi)

    km = _KMAP[_K4M]
    c1a = _rbf_host(w1[_K4G, 1, km])
    c1b = _rbf_host(w1[_K4G, 2, km])
    c1c = jnp.where(_K4G == NG - 1, 0.0, _rbf_host(w1[_K4G, 3, km]))

    w2a = _rbf_host(w2[_JG, _KMAP[0], _JR + 1])
    w2b = _rbf_host(w2[_JG, _KMAP[1], _JR + 1])
    w2c = _rbf_host(w2[_JG, _KMAP[2], _JR + 1])
    w2d = _rbf_host(w2[_JG, _KMAP[3], _JR + 1])

    m2 = jnp.einsum('gi,cj,ijk,gl,klm->gcm', rr, jnp.asarray(_E), T, rot, T,
                    precision=hi)
    bw = jnp.transpose(m2[:, :, 1:4], (0, 2, 1))
    at = bw[_JG, _JR, 0]
    bt = bw[_JG, _JR, 1]
    ct = bw[_JG, _JR, 2]

    return jnp.concatenate([
        _padq(c1a), _padq(c1b), _padq(c1c),
        _padp(w2a), _padp(w2b), _padp(w2c), _padp(w2d),
        _padp(at), _padp(bt), _padp(ct),
    ])


_E = np.zeros((3, 8), dtype=np.float32)
for _c in range(3):
    _E[_c, _c + 1] = 1.0

_ITAB = np.concatenate([
    np.pad(_I0, (0, PP - P)),
    np.pad((3 * _K4G).astype(np.int32), (0, QQ - Q)),
    np.pad((4 * _JG).astype(np.int32), (0, PP - P)),
]).astype(np.int32)

_F32 = jnp.float32
_I32 = jnp.int32

_OFF_C1 = (0, QQ, 2 * QQ)
_OFF_W2 = tuple(3 * QQ + i * PP for i in range(4))
_OFF_BW = tuple(3 * QQ + 4 * PP + i * PP for i in range(3))
_OFF_I = (0, PP, PP + QQ)


def _sc_body(n, xf, coefs, itab, mids, cents, xhat_f, idx_f, norms_o,
             c1a_r, c1b_r, c1c_r, w2a_r, w2b_r, w2c_r, w2d_r,
             at_r, bt_r, ct_r, i0_r, j0_r, k0_r,
             mids_r, cents_r, xc, xh, ub, t1b, ic, qb, nb):
    rows_per_w = n // NW
    nch = rows_per_w // R
    wid = lax.axis_index("s") * NC + lax.axis_index("c")
    row0 = wid * rows_per_w

    pltpu.sync_copy(coefs.at[pl.ds(_OFF_C1[0], QQ)], c1a_r)
    pltpu.sync_copy(coefs.at[pl.ds(_OFF_C1[1], QQ)], c1b_r)
    pltpu.sync_copy(coefs.at[pl.ds(_OFF_C1[2], QQ)], c1c_r)
    pltpu.sync_copy(coefs.at[pl.ds(_OFF_W2[0], PP)], w2a_r)
    pltpu.sync_copy(coefs.at[pl.ds(_OFF_W2[1], PP)], w2b_r)
    pltpu.sync_copy(coefs.at[pl.ds(_OFF_W2[2], PP)], w2c_r)
    pltpu.sync_copy(coefs.at[pl.ds(_OFF_W2[3], PP)], w2d_r)
    pltpu.sync_copy(coefs.at[pl.ds(_OFF_BW[0], PP)], at_r)
    pltpu.sync_copy(coefs.at[pl.ds(_OFF_BW[1], PP)], bt_r)
    pltpu.sync_copy(coefs.at[pl.ds(_OFF_BW[2], PP)], ct_r)
    pltpu.sync_copy(itab.at[pl.ds(_OFF_I[0], PP)], i0_r)
    pltpu.sync_copy(itab.at[pl.ds(_OFF_I[1], QQ)], j0_r)
    pltpu.sync_copy(itab.at[pl.ds(_OFF_I[2], PP)], k0_r)
    pltpu.sync_copy(mids, mids_r)
    pltpu.sync_copy(cents, cents_r)

    lanes = lax.iota(_I32, L)
    lane0 = lanes == 0
    m31v = plsc.load_gather(mids_r, [jnp.full((L,), 31, _I32)])
    onev = jnp.full((L,), 1, _I32)
    twov = jnp.full((L,), 2, _I32)
    threev = jnp.full((L,), 3, _I32)
    magic = jnp.full((L,), 0x5F3759DF, _I32)
    half1 = jnp.full((L,), 0x7FFF, _I32)
    maskhi = jnp.full((L,), -0x10000, _I32)
    stepv = {s: jnp.full((L,), s, _I32) for s in (32, 16, 8, 4, 2, 1)}
    probv = {s: jnp.full((L,), s - 1, _I32) for s in (16, 8, 4, 2, 1)}
    zi = jnp.zeros((L,), _I32)
    zf = jnp.zeros((L,), _F32)

    def rbf(v):
        iv = plsc.bitcast(v, _I32)
        r = iv + half1 + (lax.shift_right_logical(iv, jnp.full((L,), 16, _I32)) & onev)
        return plsc.bitcast(r & maskhi, _F32)

    ub[pl.ds(D, L)] = zf

    def chunk_body(ch, carry):
        base = row0 + ch * R
        pltpu.sync_copy(xf.at[pl.ds(base * D, R * D)], xc)

        def row_body(r, carry2):
            rb = jnp.full((L,), r * D, _I32)

            def acc_body(k2, acc):
                xv = xc[pl.ds(r * D + k2 * L, L)]
                return acc + xv * xv

            acc = lax.fori_loop(0, D // L, acc_body, jnp.zeros((L,), _F32))
            ssv = jnp.maximum(jnp.full((L,), jnp.sum(acc)), _F32(1e-16))
            iv = magic - lax.shift_right_logical(plsc.bitcast(ssv, _I32), onev)
            yv = plsc.bitcast(iv, _F32)
            nhalf = ssv * _F32(-0.5)
            for _ in range(3):
                yv = yv * (_F32(1.5) + nhalf * yv * yv)
            normv = ssv * yv
            plsc.store_scatter(nb, [jnp.full((L,), r, _I32)], normv, mask=lane0)

            def upass(j, carry3):
                off = j * L
                ub[pl.ds(off, L)] = rbf(xc[pl.ds(r * D + off, L)] / normv)
                return carry3

            lax.fori_loop(0, D // L, upass, 0)

            def stage1(j, carry3):
                off = j * L
                j0v = j0_r[pl.ds(off, L)]
                xa = plsc.load_gather(ub, [j0v])
                xb = plsc.load_gather(ub, [j0v + onev])
                xcv = plsc.load_gather(ub, [j0v + twov])
                av = c1a_r[pl.ds(off, L)]
                bv = c1b_r[pl.ds(off, L)]
                cv = c1c_r[pl.ds(off, L)]
                t1b[pl.ds(off, L)] = rbf(xa * av + xb * bv + xcv * cv)
                return carry3

            lax.fori_loop(0, QQ // L, stage1, 0)

            def stage2(j, carry3):
                off = j * L
                k0v = k0_r[pl.ds(off, L)]
                ta = plsc.load_gather(t1b, [k0v])
                tb = plsc.load_gather(t1b, [k0v + onev])
                tc = plsc.load_gather(t1b, [k0v + twov])
                td = plsc.load_gather(t1b, [k0v + threev])
                wa = w2a_r[pl.ds(off, L)]
                wb = w2b_r[pl.ds(off, L)]
                wc = w2c_r[pl.ds(off, L)]
                wd = w2d_r[pl.ds(off, L)]
                gv = ta * wa + tb * wb + tc * wc + td * wd
                cc = jnp.where(m31v < gv, stepv[32], zi)
                for s in (16, 8, 4, 2, 1):
                    pv = plsc.load_gather(mids_r, [cc + probv[s]])
                    cc = cc + jnp.where(pv < gv, stepv[s], zi)
                ic[pl.ds(r * PP + off, L)] = cc
                qb[pl.ds(off, L)] = plsc.load_gather(cents_r, [cc])
                return carry3

            lax.fori_loop(0, PP // L, stage2, 0)

            def bwd(j, carry3):
                off = j * L
                i0v = i0_r[pl.ds(off, L)]
                qa = plsc.load_gather(qb, [i0v])
                qbv = plsc.load_gather(qb, [i0v + onev])
                qcv = plsc.load_gather(qb, [i0v + twov])
                atv = at_r[pl.ds(off, L)]
                btv = bt_r[pl.ds(off, L)]
                ctv = ct_r[pl.ds(off, L)]
                xh[pl.ds(r * D + off, L)] = (qa * atv + qbv * btv + qcv * ctv) * normv
                return carry3

            lax.fori_loop(0, D // L, bwd, 0)
            return carry2

        lax.fori_loop(0, R, row_body, 0)
        pltpu.sync_copy(xh, xhat_f.at[pl.ds(base * D, R * D)])
        pltpu.sync_copy(ic, idx_f.at[pl.ds(base * PP, R * PP)])
        pltpu.sync_copy(nb, norms_o.at[pl.ds(base, R)])
        return carry

    lax.fori_loop(0, nch, chunk_body, 0)


def kernel(x, centroids_vector, rotors):
    n = x.shape[0]
    cents = centroids_vector.astype(jnp.float32)
    mids64 = jnp.concatenate(
        [(cents[:-1] + cents[1:]) * _F32(0.5), jnp.full((1,), 3e38, _F32)])
    coefs = _coef_tables(rotors)
    itab = jnp.asarray(_ITAB)
    xf = x.astype(jnp.float32).reshape(-1)

    mesh = plsc.VectorSubcoreMesh(
        core_axis_name="c", subcore_axis_name="s", num_cores=NC)
    call = functools.partial(
        pl.kernel,
        out_type=[
            jax.ShapeDtypeStruct((n * D,), jnp.float32),
            jax.ShapeDtypeStruct((n * PP,), jnp.int32),
            jax.ShapeDtypeStruct((n,), jnp.float32),
        ],
        mesh=mesh,
        compiler_params=pltpu.CompilerParams(needs_layout_passes=False),
        scratch_types=[
            pltpu.VMEM((QQ,), jnp.float32),
            pltpu.VMEM((QQ,), jnp.float32),
            pltpu.VMEM((QQ,), jnp.float32),
            pltpu.VMEM((PP,), jnp.float32),
            pltpu.VMEM((PP,), jnp.float32),
            pltpu.VMEM((PP,), jnp.float32),
            pltpu.VMEM((PP,), jnp.float32),
            pltpu.VMEM((PP,), jnp.float32),
            pltpu.VMEM((PP,), jnp.float32),
            pltpu.VMEM((PP,), jnp.float32),
            pltpu.VMEM((PP,), jnp.int32),
            pltpu.VMEM((QQ,), jnp.int32),
            pltpu.VMEM((PP,), jnp.int32),
            pltpu.VMEM((K,), jnp.float32),
            pltpu.VMEM((K,), jnp.float32),
            pltpu.VMEM((R * D,), jnp.float32),
            pltpu.VMEM((R * D,), jnp.float32),
            pltpu.VMEM((PP,), jnp.float32),
            pltpu.VMEM((QQ,), jnp.float32),
            pltpu.VMEM((R * PP,), jnp.int32),
            pltpu.VMEM((PP,), jnp.float32),
            pltpu.VMEM((R,), jnp.float32),
        ],
    )(functools.partial(_sc_body, n))

    xhat_f, idx_f, norms = call(xf, coefs, itab, mids64, cents)
    x_hat = xhat_f.reshape(n, D)
    indices = idx_f.reshape(n, PP)[:, :P].reshape(n, NG, 3)
    return (x_hat, indices, norms)
